# baseline (speedup 1.0000x reference)
import jax
import jax.numpy as jnp
from jax import lax
from jax.experimental import pallas as pl
from jax.experimental.pallas import tpu as pltpu
from jax.experimental.pallas import tpu_sc as plsc

B = 16384
EMB = 64
NC = 2
NS = 16
NW = NC * NS
NPW = B // NW
GCH = 128
NG = NPW // GCH


def _dot_body(uid_hbm, pid_hbm, utab_hbm, ptab_hbm, out_hbm,
              uidx, pidx, urows, prows, outv, sem):
    wid = lax.axis_index("s") * NC + lax.axis_index("c")
    base = wid * NPW

    pltpu.sync_copy(uid_hbm.at[pl.ds(base, NPW)], uidx)
    pltpu.sync_copy(pid_hbm.at[pl.ds(base, NPW)], pidx)

    copies = []
    for j in range(NG):
        sl = pl.ds(j * GCH, GCH)
        copies.append(pltpu.async_copy(utab_hbm.at[uidx.at[sl]], urows.at[sl], sem))
        copies.append(pltpu.async_copy(ptab_hbm.at[pidx.at[sl]], prows.at[sl], sem))
    for c in copies:
        c.wait()

    lanes = lax.iota(jnp.int32, 16)

    def group(g, carry):
        rows = g * 16 + lanes
        acc = jnp.zeros((16,), jnp.float32)
        for k in range(EMB):
            col = jnp.full((16,), k, jnp.int32)
            uv = plsc.load_gather(urows, [rows, col])
            pv = plsc.load_gather(prows, [rows, col])
            acc = acc + uv * pv
        outv[pl.ds(g * 16, 16)] = acc
        return carry

    lax.fori_loop(0, NPW // 16, group, None)
    pltpu.sync_copy(outv, out_hbm.at[pl.ds(base, NPW)])


def _build(interpret=False):
    mesh = plsc.VectorSubcoreMesh(core_axis_name="c", subcore_axis_name="s")
    return pl.kernel(
        _dot_body,
        out_type=jax.ShapeDtypeStruct((B,), jnp.float32),
        mesh=mesh,
        scratch_types=[
            pltpu.VMEM((NPW,), jnp.int32),
            pltpu.VMEM((NPW,), jnp.int32),
            pltpu.VMEM((NPW, EMB), jnp.float32),
            pltpu.VMEM((NPW, EMB), jnp.float32),
            pltpu.VMEM((NPW,), jnp.float32),
            pltpu.SemaphoreType.DMA,
        ],
        compiler_params=pltpu.CompilerParams(
            needs_layout_passes=False, use_tc_tiling_on_sc=False),
        interpret=interpret,
    )


@jax.jit
def kernel(user_ids, place_ids, user_table, place_table):
    out = _build()(user_ids.astype(jnp.int32), place_ids.astype(jnp.int32),
                   user_table, place_table)
    return out.reshape(B, 1)
